# baseline (speedup 1.0000x reference)
import functools

import jax
import jax.numpy as jnp
from jax import lax
from jax.experimental import pallas as pl
from jax.experimental.pallas import tpu as pltpu
from jax.experimental.pallas import tpu_sc as plsc

M, D, B, NCLS = 100000, 512, 16384, 1000

NC, NS = 2, 16
NW = NC * NS
SLOTS = 3136
LAST = M - (NW - 1) * SLOTS
assert SLOTS % 32 == 0 and LAST % 32 == 0 and LAST > 0
_mesh = plsc.VectorSubcoreMesh(core_axis_name="c", subcore_axis_name="s")

U_SH, U_MASK = 17, (1 << 17) - 1
W_SH, W_MASK = 14, (1 << 14) - 1


def _i32(v):
    return jnp.asarray(v, jnp.int32)


@functools.partial(
    pl.kernel,
    out_type=[
        jax.ShapeDtypeStruct((B, D), jnp.float32),
        jax.ShapeDtypeStruct((B,), jnp.int32),
        jax.ShapeDtypeStruct((M, D), jnp.float32),
        jax.ShapeDtypeStruct((M,), jnp.int32),
    ],
    mesh=_mesh,
    compiler_params=pltpu.CompilerParams(needs_layout_passes=False),
    scratch_types=[
        pltpu.VMEM((B,), jnp.int32),
        pltpu.VMEM((SLOTS,), jnp.int32),
        pltpu.VMEM((SLOTS + 16,), jnp.int32),
        pltpu.VMEM((SLOTS + 16,), jnp.int32),
        pltpu.VMEM((B + 16,), jnp.int32),
        pltpu.VMEM((B + 16,), jnp.int32),
        pltpu.VMEM((32, D), jnp.float32),
        pltpu.VMEM((32,), jnp.int32),
        pltpu.VMEM((4, 16), jnp.int32),
        pltpu.VMEM((64, D), jnp.float32),
        pltpu.VMEM((SLOTS,), jnp.int32),
        pltpu.SemaphoreType.DMA,
        pltpu.SemaphoreType.DMA,
        pltpu.SemaphoreType.DMA,
        pltpu.SemaphoreType.DMA,
        pltpu.SemaphoreType.DMA,
        pltpu.SemaphoreType.DMA,
        pltpu.SemaphoreType.DMA,
        pltpu.SemaphoreType.DMA,
        pltpu.SemaphoreType.DMA,
        pltpu.SemaphoreType.DMA,
    ],
)
def _sc_kernel(mem, mem_label, x, y, idx, retrieve_idx,
               rx, ry, nm, nl,
               idxbuf, poslocal, wslot, wpos, upk, wpk,
               rowbuf, lbuf, idxs, cbuf, lcbuf,
               sem_g, sem_s,
               sem_i0, sem_i1, sem_i2, sem_i3,
               sem_o0, sem_o1, sem_o2, sem_o3):
    cid = lax.axis_index("c")
    sid = lax.axis_index("s")
    wid = sid * NC + cid
    lo = wid * SLOTS
    is_last = wid == NW - 1
    myslots = jnp.where(is_last, _i32(LAST), _i32(SLOTS))
    hi = lo + myslots
    iota = lax.iota(jnp.int32, 16)
    zeros = iota * 0

    def bcast_at(ref, base):
        return plsc.load_gather(ref, [zeros + base])

    pltpu.sync_copy(idx, idxbuf)
    neg1 = jnp.full((16,), -1, jnp.int32)

    def init_body(v, carry):
        poslocal[pl.ds(v * 16, 16)] = neg1
        return carry
    lax.fori_loop(0, SLOTS // 16, init_body, 0, unroll=4)

    def a_body(v, carry):
        idxv = idxbuf[pl.ds(v * 16, 16)]
        posv = v * 16 + iota
        inr = (idxv >= lo) & (idxv < hi)
        tgt = jnp.clip(idxv - lo, 0, SLOTS - 1)
        plsc.store_scatter(poslocal, [tgt], posv, mask=inr)

        def w_cond(it):
            cur = plsc.load_gather(poslocal, [tgt])
            return jnp.any(inr & (cur < posv))

        def w_body(it):
            cur = plsc.load_gather(poslocal, [tgt])
            plsc.store_scatter(poslocal, [tgt], posv, mask=inr & (cur < posv))
            return it + 1
        lax.while_loop(w_cond, w_body, 0)
        return carry
    lax.fori_loop(0, B // 16, a_body, 0)

    nch = myslots // 16

    def in_cp(chunk, q, sem):
        pltpu.async_copy(mem.at[pl.ds(lo + chunk * 16, 16)],
                         cbuf.at[pl.ds(q * 16, 16)], sem)

    def out_cp(chunk, q, sem):
        pltpu.async_copy(cbuf.at[pl.ds(q * 16, 16)],
                         nm.at[pl.ds(lo + chunk * 16, 16)], sem)

    def wait_in(sem):
        pltpu.make_async_copy(mem.at[pl.ds(0, 16)],
                              cbuf.at[pl.ds(0, 16)], sem).wait()

    def wait_out(sem):
        pltpu.make_async_copy(mem.at[pl.ds(0, 16)],
                              cbuf.at[pl.ds(0, 16)], sem).wait()

    in_cp(0, 0, sem_i0)
    in_cp(1, 1, sem_i1)
    in_cp(2, 2, sem_i2)

    def quad(k, carry):
        c = 4 * k
        in_cp(c + 3, 3, sem_i3)
        wait_in(sem_i0)
        out_cp(c, 0, sem_o0)
        wait_in(sem_i1)
        out_cp(c + 1, 1, sem_o1)
        wait_in(sem_i2)
        out_cp(c + 2, 2, sem_o2)
        wait_in(sem_i3)
        out_cp(c + 3, 3, sem_o3)
        wait_out(sem_o0)

        @pl.when(c + 4 < nch)
        def _():
            in_cp(c + 4, 0, sem_i0)
        wait_out(sem_o1)

        @pl.when(c + 5 < nch)
        def _():
            in_cp(c + 5, 1, sem_i1)
        wait_out(sem_o2)

        @pl.when(c + 6 < nch)
        def _():
            in_cp(c + 6, 2, sem_i2)
        wait_out(sem_o3)
        return carry
    lax.fori_loop(0, nch // 4, quad, 0)

    @pl.when(is_last)
    def _():
        wait_in(sem_i0)
        out_cp(LAST // 16 - 2, 0, sem_o0)
        wait_in(sem_i1)
        out_cp(LAST // 16 - 1, 1, sem_o1)
        wait_out(sem_o0)
        wait_out(sem_o1)

    @pl.when(jnp.logical_not(is_last))
    def _():
        pltpu.sync_copy(mem_label.at[pl.ds(lo, SLOTS)], lcbuf)
        pltpu.sync_copy(lcbuf, nl.at[pl.ds(lo, SLOTS)])

    @pl.when(is_last)
    def _():
        pltpu.sync_copy(mem_label.at[pl.ds(lo, LAST)], lcbuf.at[pl.ds(0, LAST)])
        pltpu.sync_copy(lcbuf.at[pl.ds(0, LAST)], nl.at[pl.ds(lo, LAST)])

    def b_scan(v, cnt):
        pv = poslocal[pl.ds(v * 16, 16)]
        mask = pv >= 0
        slot_abs = lo + v * 16 + iota
        plsc.store_compressed(wslot.at[pl.ds(cnt, 16)], slot_abs, mask=mask)
        plsc.store_compressed(wpos.at[pl.ds(cnt, 16)], pv, mask=mask)
        return cnt + jnp.sum(mask.astype(jnp.int32))
    wcnt = lax.fori_loop(0, myslots // 16, b_scan, _i32(0))

    def run_pairs(cnt, decode, src_hbm, lblsrc_hbm, dstrow, dstlbl):
        npair = ((cnt + 15) // 16 + 1) // 2

        def pair(k, carry):
            d0, s0 = decode(2 * k, cnt)
            d1, s1 = decode(2 * k + 1, cnt)
            idxs[0, :] = s0
            idxs[1, :] = s1
            idxs[2, :] = d0
            idxs[3, :] = d1
            g0 = pltpu.async_copy(src_hbm.at[idxs.at[0]],
                                  rowbuf.at[pl.ds(0, 16)], sem_g)
            g1 = pltpu.async_copy(src_hbm.at[idxs.at[1]],
                                  rowbuf.at[pl.ds(16, 16)], sem_s)
            lg0 = pltpu.async_copy(lblsrc_hbm.at[idxs.at[0]],
                                   lbuf.at[pl.ds(0, 16)], sem_i0)
            lg1 = pltpu.async_copy(lblsrc_hbm.at[idxs.at[1]],
                                   lbuf.at[pl.ds(16, 16)], sem_i1)
            g0.wait()
            r0 = pltpu.async_copy(rowbuf.at[pl.ds(0, 16)],
                                  dstrow.at[idxs.at[2]], sem_o0)
            g1.wait()
            r1 = pltpu.async_copy(rowbuf.at[pl.ds(16, 16)],
                                  dstrow.at[idxs.at[3]], sem_o1)
            lg0.wait()
            l0 = pltpu.async_copy(lbuf.at[pl.ds(0, 16)],
                                  dstlbl.at[idxs.at[2]], sem_i2)
            lg1.wait()
            l1 = pltpu.async_copy(lbuf.at[pl.ds(16, 16)],
                                  dstlbl.at[idxs.at[3]], sem_i3)
            r0.wait()
            r1.wait()
            l0.wait()
            l1.wait()
            return carry
        lax.fori_loop(0, npair, pair, 0)

    @pl.when(wcnt > 0)
    def _():
        wslot[pl.ds(wcnt, 16)] = bcast_at(wslot, wcnt - 1)
        wpos[pl.ds(wcnt, 16)] = bcast_at(wpos, wcnt - 1)

        def b_decode(c, cnt):
            base = c * 16
            valid = (base + iota) < cnt
            tv = jnp.where(valid, wslot[pl.ds(base, 16)],
                           bcast_at(wslot, base))
            pv = jnp.where(valid, wpos[pl.ds(base, 16)],
                           bcast_at(wpos, base))
            return tv, pv
        run_pairs(wcnt, b_decode, x, y, nm, nl)

    pltpu.sync_copy(retrieve_idx, idxbuf)

    def c_scan(v, carry):
        ucnt, vcnt = carry
        rv = idxbuf[pl.ds(v * 16, 16)]
        inr = (rv >= lo) & (rv < hi)
        rloc = jnp.clip(rv - lo, 0, SLOTS - 1)
        pvals = plsc.load_gather(poslocal, [rloc])
        wr = inr & (pvals >= 0)
        un = inr & (pvals < 0)
        j = v * 16 + iota
        plsc.store_compressed(upk.at[pl.ds(ucnt, 16)],
                              (j << U_SH) | rv, mask=un)
        plsc.store_compressed(wpk.at[pl.ds(vcnt, 16)],
                              (j << W_SH) | pvals, mask=wr)
        return (ucnt + jnp.sum(un.astype(jnp.int32)),
                vcnt + jnp.sum(wr.astype(jnp.int32)))
    ucnt, vcnt = lax.fori_loop(0, B // 16, c_scan, (_i32(0), _i32(0)))

    @pl.when(ucnt > 0)
    def _():
        upk[pl.ds(ucnt, 16)] = bcast_at(upk, ucnt - 1)

        def u_decode(c, cnt):
            base = c * 16
            e = upk[pl.ds(base, 16)]
            e = jnp.where((base + iota) < cnt, e, bcast_at(upk, base))
            return lax.shift_right_logical(e, U_SH), e & U_MASK
        run_pairs(ucnt, u_decode, mem, mem_label, rx, ry)

    @pl.when(vcnt > 0)
    def _():
        wpk[pl.ds(vcnt, 16)] = bcast_at(wpk, vcnt - 1)

        def w_decode(c, cnt):
            base = c * 16
            e = wpk[pl.ds(base, 16)]
            e = jnp.where((base + iota) < cnt, e, bcast_at(wpk, base))
            return lax.shift_right_logical(e, W_SH), e & W_MASK
        run_pairs(vcnt, w_decode, x, y, rx, ry)


def kernel(mem, mem_label, x, y, idx, retrieve_idx):
    return tuple(_sc_kernel(mem, mem_label, x, y, idx, retrieve_idx))


# speedup vs baseline: 1.1896x; 1.1896x over previous
import functools

import jax
import jax.numpy as jnp
from jax import lax
from jax.experimental import pallas as pl
from jax.experimental.pallas import tpu as pltpu
from jax.experimental.pallas import tpu_sc as plsc

M, D, B, NCLS = 100000, 512, 16384, 1000

NC, NS = 2, 16
NW = NC * NS
SLOTS = 3136
LAST = M - (NW - 1) * SLOTS
assert SLOTS % 32 == 0 and LAST % 32 == 0 and LAST > 0
_mesh = plsc.VectorSubcoreMesh(core_axis_name="c", subcore_axis_name="s")

U_SH, U_MASK = 17, (1 << 17) - 1
W_SH, W_MASK = 14, (1 << 14) - 1


def _i32(v):
    return jnp.asarray(v, jnp.int32)


@functools.partial(
    pl.kernel,
    out_type=[
        jax.ShapeDtypeStruct((B, D), jnp.float32),
        jax.ShapeDtypeStruct((B,), jnp.int32),
        jax.ShapeDtypeStruct((M, D), jnp.float32),
        jax.ShapeDtypeStruct((M,), jnp.int32),
    ],
    mesh=_mesh,
    compiler_params=pltpu.CompilerParams(needs_layout_passes=False),
    scratch_types=[
        pltpu.VMEM((B,), jnp.int32),
        pltpu.VMEM((SLOTS,), jnp.int32),
        pltpu.VMEM((SLOTS + 16,), jnp.int32),
        pltpu.VMEM((SLOTS + 16,), jnp.int32),
        pltpu.VMEM((B + 16,), jnp.int32),
        pltpu.VMEM((B + 16,), jnp.int32),
        pltpu.VMEM((16, D), jnp.float32),
        pltpu.VMEM((16,), jnp.int32),
        pltpu.VMEM((4, 16), jnp.int32),
        pltpu.VMEM((64, D), jnp.float32),
        pltpu.VMEM((SLOTS,), jnp.int32),
        pltpu.SemaphoreType.DMA,
        pltpu.SemaphoreType.DMA,
        pltpu.SemaphoreType.DMA,
        pltpu.SemaphoreType.DMA,
        pltpu.SemaphoreType.DMA,
        pltpu.SemaphoreType.DMA,
        pltpu.SemaphoreType.DMA,
        pltpu.SemaphoreType.DMA,
        pltpu.SemaphoreType.DMA,
        pltpu.SemaphoreType.DMA,
        pltpu.SemaphoreType.DMA,
        pltpu.SemaphoreType.DMA,
    ],
)
def _sc_kernel(mem, mem_label, x, y, idx, retrieve_idx,
               rx, ry, nm, nl,
               idxbuf, poslocal, wslot, wpos, upk, wpk,
               rowbuf, lbuf, idxs, cbuf, lcbuf,
               sem_g, sem_s, sem_lg, sem_ls,
               sem_i0, sem_i1, sem_i2, sem_i3,
               sem_o0, sem_o1, sem_o2, sem_o3):
    cid = lax.axis_index("c")
    sid = lax.axis_index("s")
    wid = sid * NC + cid
    lo = wid * SLOTS
    is_last = wid == NW - 1
    myslots = jnp.where(is_last, _i32(LAST), _i32(SLOTS))
    hi = lo + myslots
    iota = lax.iota(jnp.int32, 16)
    zeros = iota * 0

    def bcast_at(ref, base):
        return plsc.load_gather(ref, [zeros + base])

    pltpu.sync_copy(idx, idxbuf)
    neg1 = jnp.full((16,), -1, jnp.int32)

    def init_body(v, carry):
        poslocal[pl.ds(v * 16, 16)] = neg1
        return carry
    lax.fori_loop(0, SLOTS // 16, init_body, 0, unroll=4)

    def a_body(v, carry):
        idxv = idxbuf[pl.ds(v * 16, 16)]
        posv = v * 16 + iota
        inr = (idxv >= lo) & (idxv < hi)
        tgt = jnp.clip(idxv - lo, 0, SLOTS - 1)
        plsc.store_scatter(poslocal, [tgt], posv, mask=inr)

        def w_cond(it):
            cur = plsc.load_gather(poslocal, [tgt])
            return jnp.any(inr & (cur < posv))

        def w_body(it):
            cur = plsc.load_gather(poslocal, [tgt])
            plsc.store_scatter(poslocal, [tgt], posv, mask=inr & (cur < posv))
            return it + 1
        lax.while_loop(w_cond, w_body, 0)
        return carry
    lax.fori_loop(0, B // 16, a_body, 0)

    pltpu.sync_copy(retrieve_idx, idxbuf)

    def c_scan(v, carry):
        ucnt, vcnt = carry
        rv = idxbuf[pl.ds(v * 16, 16)]
        inr = (rv >= lo) & (rv < hi)
        rloc = jnp.clip(rv - lo, 0, SLOTS - 1)
        pvals = plsc.load_gather(poslocal, [rloc])
        wr = inr & (pvals >= 0)
        un = inr & (pvals < 0)
        j = v * 16 + iota
        plsc.store_compressed(upk.at[pl.ds(ucnt, 16)],
                              (j << U_SH) | rv, mask=un)
        plsc.store_compressed(wpk.at[pl.ds(vcnt, 16)],
                              (j << W_SH) | pvals, mask=wr)
        return (ucnt + jnp.sum(un.astype(jnp.int32)),
                vcnt + jnp.sum(wr.astype(jnp.int32)))
    ucnt, vcnt = lax.fori_loop(0, B // 16, c_scan, (_i32(0), _i32(0)))
    nchu = (ucnt + 15) // 16
    nchw = (vcnt + 15) // 16
    ctot = nchu + nchw

    nch = myslots // 16
    nq = nch // 4

    def in_cp(chunk, q, sem):
        pltpu.async_copy(mem.at[pl.ds(lo + chunk * 16, 16)],
                         cbuf.at[pl.ds(q * 16, 16)], sem)

    def out_cp(chunk, q, sem):
        pltpu.async_copy(cbuf.at[pl.ds(q * 16, 16)],
                         nm.at[pl.ds(lo + chunk * 16, 16)], sem)

    def wait_cp(sem):
        pltpu.make_async_copy(mem.at[pl.ds(0, 16)],
                              cbuf.at[pl.ds(0, 16)], sem).wait()

    def wait_row(sem):
        pltpu.make_async_copy(mem.at[pl.ds(0, 16)], rowbuf, sem).wait()

    def wait_lbl(sem):
        pltpu.make_async_copy(mem_label.at[pl.ds(0, 16)], lbuf, sem).wait()

    in_cp(0, 0, sem_i0)
    in_cp(1, 1, sem_i1)
    in_cp(2, 2, sem_i2)
    total = jnp.maximum(nq, ctot)

    def mega(kk, carry):
        cw = kk < nq
        c = 4 * kk
        uact = kk < nchu
        wact = jnp.logical_and(nchu <= kk, kk < ctot)

        @pl.when(cw)
        def _():
            in_cp(c + 3, 3, sem_i3)

        @pl.when(uact)
        def _():
            base = kk * 16
            e = upk[pl.ds(base, 16)]
            e = jnp.where((base + iota) < ucnt, e, bcast_at(upk, base))
            idxs[0, :] = e & U_MASK
            idxs[2, :] = lax.shift_right_logical(e, U_SH)
            pltpu.async_copy(mem.at[idxs.at[0]], rowbuf, sem_g)
            pltpu.async_copy(mem_label.at[idxs.at[0]], lbuf, sem_lg)

        @pl.when(wact)
        def _():
            base = (kk - nchu) * 16
            e = wpk[pl.ds(base, 16)]
            e = jnp.where((base + iota) < vcnt, e, bcast_at(wpk, base))
            idxs[0, :] = e & W_MASK
            idxs[2, :] = lax.shift_right_logical(e, W_SH)
            pltpu.async_copy(x.at[idxs.at[0]], rowbuf, sem_g)
            pltpu.async_copy(y.at[idxs.at[0]], lbuf, sem_lg)

        @pl.when(cw)
        def _():
            wait_cp(sem_i0)
            out_cp(c, 0, sem_o0)
            wait_cp(sem_i1)
            out_cp(c + 1, 1, sem_o1)
            wait_cp(sem_i2)
            out_cp(c + 2, 2, sem_o2)
            wait_cp(sem_i3)
            out_cp(c + 3, 3, sem_o3)

        @pl.when(kk < ctot)
        def _():
            wait_row(sem_g)
            pltpu.async_copy(rowbuf, rx.at[idxs.at[2]], sem_s)
            wait_lbl(sem_lg)
            pltpu.async_copy(lbuf, ry.at[idxs.at[2]], sem_ls)
            wait_row(sem_s)
            wait_lbl(sem_ls)

        @pl.when(cw)
        def _():
            wait_cp(sem_o0)

            @pl.when(c + 4 < nch)
            def _():
                in_cp(c + 4, 0, sem_i0)
            wait_cp(sem_o1)

            @pl.when(c + 5 < nch)
            def _():
                in_cp(c + 5, 1, sem_i1)
            wait_cp(sem_o2)

            @pl.when(c + 6 < nch)
            def _():
                in_cp(c + 6, 2, sem_i2)
            wait_cp(sem_o3)
        return carry
    lax.fori_loop(0, total, mega, 0)

    @pl.when(is_last)
    def _():
        wait_cp(sem_i0)
        out_cp(LAST // 16 - 2, 0, sem_o0)
        wait_cp(sem_i1)
        out_cp(LAST // 16 - 1, 1, sem_o1)
        wait_cp(sem_o0)
        wait_cp(sem_o1)

    @pl.when(jnp.logical_not(is_last))
    def _():
        pltpu.sync_copy(mem_label.at[pl.ds(lo, SLOTS)], lcbuf)
        pltpu.sync_copy(lcbuf, nl.at[pl.ds(lo, SLOTS)])

    @pl.when(is_last)
    def _():
        pltpu.sync_copy(mem_label.at[pl.ds(lo, LAST)], lcbuf.at[pl.ds(0, LAST)])
        pltpu.sync_copy(lcbuf.at[pl.ds(0, LAST)], nl.at[pl.ds(lo, LAST)])

    def b_scan(v, cnt):
        pv = poslocal[pl.ds(v * 16, 16)]
        mask = pv >= 0
        slot_abs = lo + v * 16 + iota
        plsc.store_compressed(wslot.at[pl.ds(cnt, 16)], slot_abs, mask=mask)
        plsc.store_compressed(wpos.at[pl.ds(cnt, 16)], pv, mask=mask)
        return cnt + jnp.sum(mask.astype(jnp.int32))
    wcnt = lax.fori_loop(0, myslots // 16, b_scan, _i32(0))

    def b_rows(cc, carry):
        base = cc * 16
        tv = wslot[pl.ds(base, 16)]
        pv = wpos[pl.ds(base, 16)]
        valid = (base + iota) < wcnt
        tgt = jnp.where(valid, tv, bcast_at(wslot, base))
        src = jnp.where(valid, pv, bcast_at(wpos, base))
        pltpu.async_copy(x.at[src], rowbuf, sem_g).wait()
        pltpu.async_copy(rowbuf, nm.at[tgt], sem_s).wait()
        pltpu.async_copy(y.at[src], lbuf, sem_g).wait()
        pltpu.async_copy(lbuf, nl.at[tgt], sem_s).wait()
        return carry
    lax.fori_loop(0, (wcnt + 15) // 16, b_rows, 0)


def kernel(mem, mem_label, x, y, idx, retrieve_idx):
    return tuple(_sc_kernel(mem, mem_label, x, y, idx, retrieve_idx))
